# Initial kernel scaffold; baseline (speedup 1.0000x reference)
import jax
import jax.numpy as jnp
from jax.experimental import pallas as pl


def kernel(x, table, W, b):
    raise NotImplementedError("write your pallas kernel here")


# speedup vs baseline: 2.2803x; 2.2803x over previous
import functools

import jax
import jax.numpy as jnp
from jax import lax
from jax.experimental import pallas as pl
from jax.experimental.pallas import tpu as pltpu
from jax.experimental.pallas import tpu_sc as plsc

_VOCAB = 1000000
_EMBED = 32
_NUM_CLASSES = 50
_PAD_IDX = 0
_BATCH = 4096
_SEQ = 200

_NC = 2
_NS = 16
_NW = _NC * _NS
_B_PER_W = _BATCH // _NW
_C0, _C1 = 120, 80
_LANES = 16

_mesh = plsc.VectorSubcoreMesh(core_axis_name="c", subcore_axis_name="s")


@functools.partial(
    pl.kernel,
    mesh=_mesh,
    out_type=jax.ShapeDtypeStruct((_BATCH, _EMBED), jnp.float32),
    compiler_params=pltpu.CompilerParams(use_tc_tiling_on_sc=False),
    scratch_types=[
        pltpu.VMEM((_B_PER_W * _SEQ,), jnp.int32),
        pltpu.VMEM((_SEQ, _EMBED), jnp.float32),
        pltpu.VMEM((_SEQ, _EMBED), jnp.float32),
        pltpu.VMEM((_B_PER_W, _EMBED), jnp.float32),
        pltpu.SemaphoreType.DMA,
        pltpu.SemaphoreType.DMA,
    ],
)
def _sc_sum(x_hbm, table_hbm, out_hbm, idx_v, buf0, buf1, sums_v, sem0, sem1):
    wid = lax.axis_index("s") * _NC + lax.axis_index("c")
    base = wid * _B_PER_W

    pltpu.sync_copy(x_hbm.at[pl.ds(base * _SEQ, _B_PER_W * _SEQ)], idx_v)

    bufs = (buf0, buf1)
    sems = (sem0, sem1)

    def issue(r, buf, sem):
        off = pl.multiple_of(r * _SEQ, 8)
        pltpu.async_copy(table_hbm.at[idx_v.at[pl.ds(off, _C0)]],
                         buf.at[pl.ds(0, _C0)], sem)
        pltpu.async_copy(table_hbm.at[idx_v.at[pl.ds(off + _C0, _C1)]],
                         buf.at[pl.ds(_C0, _C1)], sem)

    def wait(r, buf, sem):
        off = pl.multiple_of(r * _SEQ, 8)
        pltpu.make_async_copy(table_hbm.at[idx_v.at[pl.ds(off, _C0)]],
                              buf.at[pl.ds(0, _C0)], sem).wait()
        pltpu.make_async_copy(table_hbm.at[idx_v.at[pl.ds(off + _C0, _C1)]],
                              buf.at[pl.ds(_C0, _C1)], sem).wait()

    issue(0, buf0, sem0)

    def outer(i, carry):
        for p in range(2):
            r = i * 2 + p
            nxt = r + 1

            @pl.when(nxt < _B_PER_W)
            def _():
                issue(nxt, bufs[1 - p], sems[1 - p])

            wait(r, bufs[p], sems[p])
            buf = bufs[p]

            def accum(j, acc):
                a0, a1 = acc
                for u in range(8):
                    row = j * 8 + u
                    a0 = a0 + buf[row, pl.ds(0, _LANES)]
                    a1 = a1 + buf[row, pl.ds(_LANES, _LANES)]
                return (a0, a1)

            zero = jnp.zeros((_LANES,), jnp.float32)
            a0, a1 = lax.fori_loop(0, _SEQ // 8, accum, (zero, zero))
            sums_v[r, pl.ds(0, _LANES)] = a0
            sums_v[r, pl.ds(_LANES, _LANES)] = a1
        return carry

    lax.fori_loop(0, _B_PER_W // 2, outer, 0)
    pltpu.sync_copy(sums_v, out_hbm.at[pl.ds(base, _B_PER_W)])


def _tc_head(summed_ref, x_ref, w_ref, b_ref, out_ref):
    xb = x_ref[...]
    cnt = jnp.sum((xb != _PAD_IDX).astype(jnp.float32), axis=1, keepdims=True)
    cnt = jnp.maximum(cnt, 1.0)
    avg = summed_ref[...] / cnt
    out_ref[...] = (
        jnp.dot(avg, w_ref[...], preferred_element_type=jnp.float32)
        + b_ref[...]
    )


_TC_BLK = 512


def kernel(x, table, W, b):
    xflat = x.reshape(-1)
    summed = _sc_sum(xflat, table)
    grid = (_BATCH // _TC_BLK,)
    out = pl.pallas_call(
        _tc_head,
        grid=grid,
        in_specs=[
            pl.BlockSpec((_TC_BLK, _EMBED), lambda i: (i, 0)),
            pl.BlockSpec((_TC_BLK, _SEQ), lambda i: (i, 0)),
            pl.BlockSpec((_EMBED, _NUM_CLASSES), lambda i: (0, 0)),
            pl.BlockSpec((1, _NUM_CLASSES), lambda i: (0, 0)),
        ],
        out_specs=pl.BlockSpec((_TC_BLK, _NUM_CLASSES), lambda i: (i, 0)),
        out_shape=jax.ShapeDtypeStruct((_BATCH, _NUM_CLASSES), jnp.float32),
    )(summed, x, W, b.reshape(1, _NUM_CLASSES))
    return out
